# baseline (speedup 1.0000x reference)
import functools

import jax
import jax.numpy as jnp
from jax import lax
from jax.experimental import pallas as pl
from jax.experimental.pallas import tpu as pltpu
from jax.experimental.pallas import tpu_sc as plsc

N = 10000
E = 320000
D = 128

NC = 2
NS = 16
L = 16
NW = NC * NS
CHUNK = 80
N_CHUNKS = E // CHUNK
STEPS = -(-N_CHUNKS // NW)
N_PAD = 10240
ROWS_PER_TILE = N_PAD // NS

_DNUMS = lax.GatherDimensionNumbers(
    offset_dims=(), collapsed_slice_dims=(0,), start_index_map=(0,))


def _pack_words(x):
    xq = x.astype(jnp.bfloat16).astype(jnp.float32)
    bits = lax.bitcast_convert_type(xq, jnp.int32)
    lo = jnp.concatenate(
        [bits[:, 32 * g:32 * g + 16] for g in range(D // 32)], axis=1)
    hi = jnp.concatenate(
        [bits[:, 32 * g + 16:32 * g + 32] for g in range(D // 32)], axis=1)
    return jnp.bitwise_or(
        jnp.bitwise_and(lax.shift_right_logical(lo, 16), jnp.int32(0xFFFF)),
        jnp.bitwise_and(hi, jnp.int32(-65536)))


def _prep_bf16(x):
    BR = 1000

    def body(x_ref, o_ref):
        o_ref[...] = _pack_words(x_ref[...])

    spec = pl.BlockSpec((BR, D), lambda i: (i, 0))
    ospec = pl.BlockSpec((BR, D // 2), lambda i: (i, 0))
    return pl.pallas_call(
        body, grid=(N // BR,), in_specs=[spec], out_specs=ospec,
        out_shape=jax.ShapeDtypeStruct((N, D // 2), jnp.int32))(x)


def _spmm_sc(x_bf, src, dst, vals):
    mesh = plsc.VectorSubcoreMesh(core_axis_name="c", subcore_axis_name="s")

    @functools.partial(
        pl.kernel,
        mesh=mesh,
        compiler_params=pltpu.CompilerParams(use_tc_tiling_on_sc=False),
        out_type=jax.ShapeDtypeStruct((2 * N_PAD, D), jnp.float32),
        scratch_types=(
            [pltpu.VMEM((CHUNK,), jnp.int32) for _ in range(4)]
            + [pltpu.VMEM((CHUNK,), jnp.int32) for _ in range(4)]
            + [pltpu.VMEM((CHUNK,), jnp.float32) for _ in range(4)]
            + [pltpu.VMEM((CHUNK, D // 2), jnp.int32) for _ in range(2)]
            + [pltpu.VMEM((CHUNK, D), jnp.float32) for _ in range(2)]
            + [pltpu.VMEM_SHARED((N_PAD, D), jnp.float32)]
            + [pltpu.SemaphoreType.DMA for _ in range(12)]
        ),
    )
    def spmm_kernel(x_hbm, src_hbm, dst_hbm, val_hbm, out_hbm,
                    s0, s1, s2, s3, d0, d1, d2, d3, v0, v1, v2, v3,
                    rb0, rb1, rf0, rf1, acc_sh,
                    is0, is1, is2, is3, ds0, ds1, ds2, ds3,
                    gs0, gs1, cs0, cs1):
        srcb = [s0, s1, s2, s3]
        dstb = [d0, d1, d2, d3]
        valb = [v0, v1, v2, v3]
        rowsb = [rb0, rb1]
        rowsf = [rf0, rf1]
        isem = [is0, is1, is2, is3]
        dsem = [ds0, ds1, ds2, ds3]
        gsem = [gs0, gs1]
        csem = [cs0, cs1]

        cid = lax.axis_index("c")
        sid = lax.axis_index("s")
        wid = sid * NC + cid

        def issue_sv(chunk, i):
            base = chunk * CHUNK
            pltpu.make_async_copy(
                src_hbm.at[pl.ds(base, CHUNK)], srcb[i], isem[i]).start()
            pltpu.make_async_copy(
                val_hbm.at[pl.ds(base, CHUNK)], valb[i], isem[i]).start()

        def wait_sv(i):
            pltpu.make_async_copy(
                src_hbm.at[pl.ds(0, CHUNK)], srcb[i], isem[i]).wait()
            pltpu.make_async_copy(
                val_hbm.at[pl.ds(0, CHUNK)], valb[i], isem[i]).wait()

        def issue_dst(chunk, i):
            pltpu.make_async_copy(
                dst_hbm.at[pl.ds(chunk * CHUNK, CHUNK)], dstb[i],
                dsem[i]).start()

        def wait_dst(i):
            pltpu.make_async_copy(
                dst_hbm.at[pl.ds(0, CHUNK)], dstb[i], dsem[i]).wait()

        def unpack_scale(r, i):
            def group_body(g, cc):
                vals16 = valb[i][pl.ds(g * L, L)]
                for q in range(L):
                    v = lax.gather(
                        vals16, jnp.full((L, 1), q, jnp.int32), _DNUMS,
                        slice_sizes=(1,),
                        mode=lax.GatherScatterMode.PROMISE_IN_BOUNDS)
                    row = g * L + q
                    for j in range(D // 32):
                        wi = rowsb[r][row, pl.ds(L * j, L)]
                        a = lax.bitcast_convert_type(
                            jnp.left_shift(wi, 16), jnp.float32)
                        b = lax.bitcast_convert_type(
                            jnp.bitwise_and(wi, jnp.int32(-65536)),
                            jnp.float32)
                        rowsf[r][row, pl.ds(32 * j, L)] = a * v
                        rowsf[r][row, pl.ds(32 * j + L, L)] = b * v
                return cc
            lax.fori_loop(0, CHUNK // L, group_body, 0)

        def zero_row(i, c):
            for j in range(D // L):
                rf0[i, pl.ds(j * L, L)] = jnp.zeros((L,), jnp.float32)
            return c
        lax.fori_loop(0, CHUNK, zero_row, 0)
        for q in range(ROWS_PER_TILE // CHUNK):
            pltpu.sync_copy(
                rf0, acc_sh.at[pl.ds(sid * ROWS_PER_TILE + q * CHUNK, CHUNK)])
        plsc.subcore_barrier()

        issue_sv(wid, 0)
        issue_dst(wid, 0)
        issue_sv(NW + wid, 1)
        issue_dst(NW + wid, 1)
        wait_sv(0)
        pltpu.make_async_copy(x_hbm.at[srcb[0]], rowsb[0], gsem[0]).start()

        def super_step(u, c):
            for k in range(4):
                t = 4 * u + k
                b2, b4 = k % 2, k % 4
                n2, n4 = (k + 1) % 2, (k + 1) % 4
                p4 = (k + 2) % 4
                cur = t * NW + wid
                nxt = cur + NW
                pre = cur + 2 * NW

                @pl.when(nxt < N_CHUNKS)
                def _():
                    @pl.when(t >= 1)
                    def _():
                        pltpu.make_async_copy(
                            rowsf[n2], acc_sh.at[dstb[n4]], csem[n2]).wait()
                    wait_sv(n4)
                    pltpu.make_async_copy(
                        x_hbm.at[srcb[n4]], rowsb[n2], gsem[n2]).start()

                @pl.when(cur < N_CHUNKS)
                def _():
                    pltpu.make_async_copy(
                        x_hbm.at[srcb[b4]], rowsb[b2], gsem[b2]).wait()
                    unpack_scale(b2, b4)
                    wait_dst(b4)
                    pltpu.make_async_copy(
                        rowsf[b2], acc_sh.at[dstb[b4]], csem[b2]).start(
                            add=True)

                @pl.when(pre < N_CHUNKS)
                def _():
                    issue_sv(pre, p4)
                    issue_dst(pre, p4)
            return c
        lax.fori_loop(0, (STEPS + 3) // 4, super_step, 0)

        for i in range(2):
            pltpu.make_async_copy(rowsf[i], acc_sh.at[dstb[i]], csem[i]).wait()

        plsc.subcore_barrier()
        pltpu.sync_copy(
            acc_sh.at[pl.ds(sid * ROWS_PER_TILE, ROWS_PER_TILE)],
            out_hbm.at[pl.ds(cid * N_PAD + sid * ROWS_PER_TILE, ROWS_PER_TILE)])

    return spmm_kernel(x_bf, src, dst, vals)


def _dense_layer(ego, s0, s1, W1, b1, W2, b2, emit_bf):
    BR = 1000

    def body(ego_ref, s0_ref, s1_ref, W1_ref, b1_ref, W2_ref, b2_ref,
             *out_refs):
        ego_b = ego_ref[...]
        side = s0_ref[...] + s1_ref[...]
        a = jnp.dot(ego_b + side, W1_ref[...],
                    preferred_element_type=jnp.float32) + b1_ref[...]
        sum_emb = jnp.where(a >= 0, a, 0.01 * a)
        b = jnp.dot(ego_b * side, W2_ref[...],
                    preferred_element_type=jnp.float32) + b2_ref[...]
        bi_emb = jnp.where(b >= 0, b, 0.01 * b)
        e = sum_emb + bi_emb
        out_refs[0][...] = e
        n = jnp.sqrt(jnp.sum(e * e, axis=-1, keepdims=True))
        no = e / jnp.maximum(n, 1e-12)
        out_refs[1][...] = no
        if emit_bf:
            out_refs[2][...] = _pack_words(no)

    row_spec = pl.BlockSpec((BR, D), lambda i: (i, 0))
    mat_spec = pl.BlockSpec((D, D), lambda i: (0, 0))
    vec_spec = pl.BlockSpec((1, D), lambda i: (0, 0))
    out_specs = [row_spec, row_spec]
    out_shape = [jax.ShapeDtypeStruct((N, D), jnp.float32)] * 2
    if emit_bf:
        out_specs.append(pl.BlockSpec((BR, D // 2), lambda i: (i, 0)))
        out_shape.append(jax.ShapeDtypeStruct((N, D // 2), jnp.int32))
    return pl.pallas_call(
        body,
        grid=(N // BR,),
        in_specs=[row_spec, row_spec, row_spec,
                  mat_spec, vec_spec, mat_spec, vec_spec],
        out_specs=out_specs,
        out_shape=out_shape,
    )(ego, s0, s1, W1, b1.reshape(1, D), W2, b2.reshape(1, D))


def kernel(embeddings, edge_index, edge_values,
           W1_0, b1_0, W2_0, b2_0, W1_1, b1_1, W2_1, b2_1):
    src = edge_index[0].astype(jnp.int32)
    dst = edge_index[1].astype(jnp.int32)
    vals = edge_values.astype(jnp.float32)

    emb_bf = _prep_bf16(embeddings)
    parts0 = _spmm_sc(emb_bf, src, dst, vals)
    ego1, norm1, norm1_bf = _dense_layer(
        embeddings, parts0[:N], parts0[N_PAD:N_PAD + N],
        W1_0, b1_0, W2_0, b2_0, emit_bf=True)

    parts1 = _spmm_sc(norm1_bf, src, dst, vals)
    _, norm2 = _dense_layer(
        ego1, parts1[:N], parts1[N_PAD:N_PAD + N],
        W1_1, b1_1, W2_1, b2_1, emit_bf=False)
    return (embeddings, norm1, norm2)


# speedup vs baseline: 2.2399x; 2.2399x over previous
import functools

import jax
import jax.numpy as jnp
from jax import lax
from jax.experimental import pallas as pl
from jax.experimental.pallas import tpu as pltpu
from jax.experimental.pallas import tpu_sc as plsc

N = 10000
E = 320000
D = 128

NC = 2
NS = 16
L = 16
NW = NC * NS
CHUNK = 128
N_CHUNKS = E // CHUNK
STEPS = -(-N_CHUNKS // NW)
N_PAD = 10240
ROWS_PER_TILE = N_PAD // NS

_DNUMS = lax.GatherDimensionNumbers(
    offset_dims=(), collapsed_slice_dims=(0,), start_index_map=(0,))


def _spmm_sc(x, src, dst, vals):
    mesh = plsc.VectorSubcoreMesh(core_axis_name="c", subcore_axis_name="s")

    @functools.partial(
        pl.kernel,
        mesh=mesh,
        out_type=jax.ShapeDtypeStruct((2, N_PAD, D), jnp.float32),
        scratch_types=(
            [pltpu.VMEM((CHUNK,), jnp.int32) for _ in range(4)]
            + [pltpu.VMEM((CHUNK,), jnp.int32) for _ in range(4)]
            + [pltpu.VMEM((CHUNK,), jnp.float32) for _ in range(4)]
            + [pltpu.VMEM((CHUNK, D), jnp.float32) for _ in range(2)]
            + [pltpu.VMEM_SHARED((N_PAD, D), jnp.float32)]
            + [pltpu.SemaphoreType.DMA for _ in range(12)]
        ),
    )
    def spmm_kernel(x_hbm, src_hbm, dst_hbm, val_hbm, out_hbm,
                    s0, s1, s2, s3, d0, d1, d2, d3, v0, v1, v2, v3,
                    r0, r1, acc_sh,
                    is0, is1, is2, is3, ds0, ds1, ds2, ds3,
                    gs0, gs1, cs0, cs1):
        srcb = [s0, s1, s2, s3]
        dstb = [d0, d1, d2, d3]
        valb = [v0, v1, v2, v3]
        rows = [r0, r1]
        isem = [is0, is1, is2, is3]
        dsem = [ds0, ds1, ds2, ds3]
        gsem = [gs0, gs1]
        csem = [cs0, cs1]

        cid = lax.axis_index("c")
        sid = lax.axis_index("s")
        wid = sid * NC + cid

        def issue_sv(chunk, i):
            base = chunk * CHUNK
            pltpu.make_async_copy(
                src_hbm.at[pl.ds(base, CHUNK)], srcb[i], isem[i]).start()
            pltpu.make_async_copy(
                val_hbm.at[pl.ds(base, CHUNK)], valb[i], isem[i]).start()

        def wait_sv(i):
            pltpu.make_async_copy(
                src_hbm.at[pl.ds(0, CHUNK)], srcb[i], isem[i]).wait()
            pltpu.make_async_copy(
                val_hbm.at[pl.ds(0, CHUNK)], valb[i], isem[i]).wait()

        def issue_dst(chunk, i):
            pltpu.make_async_copy(
                dst_hbm.at[pl.ds(chunk * CHUNK, CHUNK)], dstb[i],
                dsem[i]).start()

        def wait_dst(i):
            pltpu.make_async_copy(
                dst_hbm.at[pl.ds(0, CHUNK)], dstb[i], dsem[i]).wait()

        def scale_rows(r, i):
            def group_body(g, cc):
                vals16 = valb[i][pl.ds(g * L, L)]
                for q in range(L):
                    v = lax.gather(
                        vals16, jnp.full((L, 1), q, jnp.int32), _DNUMS,
                        slice_sizes=(1,),
                        mode=lax.GatherScatterMode.PROMISE_IN_BOUNDS)
                    row = g * L + q
                    for j in range(D // L):
                        rows[r][row, pl.ds(j * L, L)] = (
                            rows[r][row, pl.ds(j * L, L)] * v)
                return cc
            lax.fori_loop(0, CHUNK // L, group_body, 0)

        def zero_row(i, c):
            for j in range(D // L):
                r0[i, pl.ds(j * L, L)] = jnp.zeros((L,), jnp.float32)
            return c
        lax.fori_loop(0, CHUNK, zero_row, 0)
        for q in range(ROWS_PER_TILE // CHUNK):
            pltpu.sync_copy(
                r0, acc_sh.at[pl.ds(sid * ROWS_PER_TILE + q * CHUNK, CHUNK)])
        plsc.subcore_barrier()

        issue_sv(wid, 0)
        issue_dst(wid, 0)
        issue_sv(NW + wid, 1)
        issue_dst(NW + wid, 1)
        wait_sv(0)
        pltpu.make_async_copy(x_hbm.at[srcb[0]], rows[0], gsem[0]).start()

        def super_step(u, c):
            for k in range(4):
                t = 4 * u + k
                b2, b4 = k % 2, k % 4
                n2, n4 = (k + 1) % 2, (k + 1) % 4
                p4 = (k + 2) % 4
                cur = t * NW + wid
                nxt = cur + NW
                pre = cur + 2 * NW

                @pl.when(nxt < N_CHUNKS)
                def _():
                    @pl.when(t >= 1)
                    def _():
                        pltpu.make_async_copy(
                            rows[n2], acc_sh.at[dstb[n4]], csem[n2]).wait()
                    wait_sv(n4)
                    pltpu.make_async_copy(
                        x_hbm.at[srcb[n4]], rows[n2], gsem[n2]).start()

                @pl.when(cur < N_CHUNKS)
                def _():
                    pltpu.make_async_copy(
                        x_hbm.at[srcb[b4]], rows[b2], gsem[b2]).wait()
                    scale_rows(b2, b4)
                    wait_dst(b4)
                    pltpu.make_async_copy(
                        rows[b2], acc_sh.at[dstb[b4]], csem[b2]).start(
                            add=True)

                @pl.when(pre < N_CHUNKS)
                def _():
                    issue_sv(pre, p4)
                    issue_dst(pre, p4)
            return c
        lax.fori_loop(0, (STEPS + 3) // 4, super_step, 0)

        for i in range(2):
            pltpu.make_async_copy(rows[i], acc_sh.at[dstb[i]], csem[i]).wait()

        plsc.subcore_barrier()
        pltpu.sync_copy(
            acc_sh.at[pl.ds(sid * ROWS_PER_TILE, ROWS_PER_TILE)],
            out_hbm.at[cid, pl.ds(sid * ROWS_PER_TILE, ROWS_PER_TILE)])

    return spmm_kernel(x, src, dst, vals)


def _dense_layer(ego, parts, W1, b1, W2, b2):
    BR = 1000

    def body(ego_ref, s0_ref, s1_ref, W1_ref, b1_ref, W2_ref, b2_ref,
             eo_ref, no_ref):
        ego_b = ego_ref[...]
        side = s0_ref[0] + s1_ref[0]
        a = jnp.dot(ego_b + side, W1_ref[...],
                    preferred_element_type=jnp.float32) + b1_ref[...]
        sum_emb = jnp.where(a >= 0, a, 0.01 * a)
        b = jnp.dot(ego_b * side, W2_ref[...],
                    preferred_element_type=jnp.float32) + b2_ref[...]
        bi_emb = jnp.where(b >= 0, b, 0.01 * b)
        e = sum_emb + bi_emb
        eo_ref[...] = e
        n = jnp.sqrt(jnp.sum(e * e, axis=-1, keepdims=True))
        no_ref[...] = e / jnp.maximum(n, 1e-12)

    row_spec = pl.BlockSpec((BR, D), lambda i: (i, 0))
    p0_spec = pl.BlockSpec((1, BR, D), lambda i: (0, i, 0))
    p1_spec = pl.BlockSpec((1, BR, D), lambda i: (1, i, 0))
    mat_spec = pl.BlockSpec((D, D), lambda i: (0, 0))
    vec_spec = pl.BlockSpec((1, D), lambda i: (0, 0))
    return pl.pallas_call(
        body,
        grid=(N // BR,),
        in_specs=[row_spec, p0_spec, p1_spec,
                  mat_spec, vec_spec, mat_spec, vec_spec],
        out_specs=[row_spec, row_spec],
        out_shape=[jax.ShapeDtypeStruct((N, D), jnp.float32)] * 2,
    )(ego, parts, parts, W1, b1.reshape(1, D), W2, b2.reshape(1, D))


def kernel(embeddings, edge_index, edge_values,
           W1_0, b1_0, W2_0, b2_0, W1_1, b1_1, W2_1, b2_1):
    src = edge_index[0].astype(jnp.int32)
    dst = edge_index[1].astype(jnp.int32)
    vals = edge_values.astype(jnp.float32)

    parts0 = _spmm_sc(embeddings, src, dst, vals)
    ego1, norm1 = _dense_layer(embeddings, parts0, W1_0, b1_0, W2_0, b2_0)
    parts1 = _spmm_sc(norm1, src, dst, vals)
    _, norm2 = _dense_layer(ego1, parts1, W1_1, b1_1, W2_1, b2_1)
    return (embeddings, norm1, norm2)


# speedup vs baseline: 2.2476x; 1.0034x over previous
import functools

import jax
import jax.numpy as jnp
from jax import lax
from jax.experimental import pallas as pl
from jax.experimental.pallas import tpu as pltpu
from jax.experimental.pallas import tpu_sc as plsc

N = 10000
E = 320000
D = 128

NC = 2
NS = 16
L = 16
NW = NC * NS
CHUNK = 128
N_CHUNKS = E // CHUNK
STEPS = -(-N_CHUNKS // NW)
N_PAD = 10240
ROWS_PER_TILE = N_PAD // NS

_DNUMS = lax.GatherDimensionNumbers(
    offset_dims=(), collapsed_slice_dims=(0,), start_index_map=(0,))


def _spmm_sc(x, src, dst, vals):
    mesh = plsc.VectorSubcoreMesh(core_axis_name="c", subcore_axis_name="s")

    @functools.partial(
        pl.kernel,
        mesh=mesh,
        out_type=jax.ShapeDtypeStruct((2, N_PAD, D), jnp.float32),
        scratch_types=(
            [pltpu.VMEM((CHUNK,), jnp.int32) for _ in range(4)]
            + [pltpu.VMEM((CHUNK,), jnp.int32) for _ in range(4)]
            + [pltpu.VMEM((CHUNK,), jnp.float32) for _ in range(4)]
            + [pltpu.VMEM((CHUNK, D), jnp.float32) for _ in range(2)]
            + [pltpu.VMEM_SHARED((N_PAD, D), jnp.float32)]
            + [pltpu.SemaphoreType.DMA for _ in range(12)]
        ),
    )
    def spmm_kernel(x_hbm, src_hbm, dst_hbm, val_hbm, out_hbm,
                    s0, s1, s2, s3, d0, d1, d2, d3, v0, v1, v2, v3,
                    r0, r1, acc_sh,
                    is0, is1, is2, is3, ds0, ds1, ds2, ds3,
                    gs0, gs1, cs0, cs1):
        srcb = [s0, s1, s2, s3]
        dstb = [d0, d1, d2, d3]
        valb = [v0, v1, v2, v3]
        rows = [r0, r1]
        isem = [is0, is1, is2, is3]
        dsem = [ds0, ds1, ds2, ds3]
        gsem = [gs0, gs1]
        csem = [cs0, cs1]

        cid = lax.axis_index("c")
        sid = lax.axis_index("s")
        wid = sid * NC + cid

        def issue_sv(chunk, i):
            base = chunk * CHUNK
            pltpu.make_async_copy(
                src_hbm.at[pl.ds(base, CHUNK)], srcb[i], isem[i]).start()
            pltpu.make_async_copy(
                val_hbm.at[pl.ds(base, CHUNK)], valb[i], isem[i]).start()

        def wait_sv(i):
            pltpu.make_async_copy(
                src_hbm.at[pl.ds(0, CHUNK)], srcb[i], isem[i]).wait()
            pltpu.make_async_copy(
                val_hbm.at[pl.ds(0, CHUNK)], valb[i], isem[i]).wait()

        def issue_dst(chunk, i):
            pltpu.make_async_copy(
                dst_hbm.at[pl.ds(chunk * CHUNK, CHUNK)], dstb[i],
                dsem[i]).start()

        def wait_dst(i):
            pltpu.make_async_copy(
                dst_hbm.at[pl.ds(0, CHUNK)], dstb[i], dsem[i]).wait()

        def scale_rows(r, i):
            def group_body(g, cc):
                vals16 = valb[i][pl.ds(g * L, L)]
                for q in range(L):
                    v = lax.gather(
                        vals16, jnp.full((L, 1), q, jnp.int32), _DNUMS,
                        slice_sizes=(1,),
                        mode=lax.GatherScatterMode.PROMISE_IN_BOUNDS)
                    row = g * L + q
                    for j in range(D // L):
                        rows[r][row, pl.ds(j * L, L)] = (
                            rows[r][row, pl.ds(j * L, L)] * v)
                return cc
            lax.fori_loop(0, CHUNK // L, group_body, 0)

        def zero_row(i, c):
            for j in range(D // L):
                r0[i, pl.ds(j * L, L)] = jnp.zeros((L,), jnp.float32)
            return c
        lax.fori_loop(0, CHUNK, zero_row, 0)
        for q in range(ROWS_PER_TILE // CHUNK):
            pltpu.sync_copy(
                r0, acc_sh.at[pl.ds(sid * ROWS_PER_TILE + q * CHUNK, CHUNK)])
        plsc.subcore_barrier()

        issue_sv(wid, 0)
        issue_dst(wid, 0)
        issue_sv(NW + wid, 1)
        issue_dst(NW + wid, 1)
        wait_sv(0)
        pltpu.make_async_copy(
            x_hbm.at[srcb[0].at[pl.ds(0, CHUNK // 2)]],
            rows[0].at[pl.ds(0, CHUNK // 2)], gsem[0]).start()
        pltpu.make_async_copy(
            x_hbm.at[srcb[0].at[pl.ds(CHUNK // 2, CHUNK // 2)]],
            rows[0].at[pl.ds(CHUNK // 2, CHUNK // 2)], gsem[0]).start()

        def super_step(u, c):
            for k in range(4):
                t = 4 * u + k
                b2, b4 = k % 2, k % 4
                n2, n4 = (k + 1) % 2, (k + 1) % 4
                p4 = (k + 2) % 4
                cur = t * NW + wid
                nxt = cur + NW
                pre = cur + 2 * NW

                @pl.when(nxt < N_CHUNKS)
                def _():
                    @pl.when(t >= 1)
                    def _():
                        pltpu.make_async_copy(
                            rows[n2], acc_sh.at[dstb[n4]], csem[n2]).wait()
                    wait_sv(n4)
                    pltpu.make_async_copy(
                        x_hbm.at[srcb[n4].at[pl.ds(0, CHUNK // 2)]],
                        rows[n2].at[pl.ds(0, CHUNK // 2)], gsem[n2]).start()
                    pltpu.make_async_copy(
                        x_hbm.at[srcb[n4].at[pl.ds(CHUNK // 2, CHUNK // 2)]],
                        rows[n2].at[pl.ds(CHUNK // 2, CHUNK // 2)],
                        gsem[n2]).start()

                @pl.when(cur < N_CHUNKS)
                def _():
                    for hh in range(2):
                        pltpu.make_async_copy(
                            x_hbm.at[srcb[b4].at[pl.ds(0, CHUNK // 2)]],
                            rows[b2].at[pl.ds(0, CHUNK // 2)],
                            gsem[b2]).wait()
                    scale_rows(b2, b4)
                    wait_dst(b4)
                    pltpu.make_async_copy(
                        rows[b2], acc_sh.at[dstb[b4]], csem[b2]).start(
                            add=True)

                @pl.when(pre < N_CHUNKS)
                def _():
                    issue_sv(pre, p4)
                    issue_dst(pre, p4)
            return c
        lax.fori_loop(0, (STEPS + 3) // 4, super_step, 0)

        for i in range(2):
            pltpu.make_async_copy(rows[i], acc_sh.at[dstb[i]], csem[i]).wait()

        plsc.subcore_barrier()
        pltpu.sync_copy(
            acc_sh.at[pl.ds(sid * ROWS_PER_TILE, ROWS_PER_TILE)],
            out_hbm.at[cid, pl.ds(sid * ROWS_PER_TILE, ROWS_PER_TILE)])

    return spmm_kernel(x, src, dst, vals)


def _dense_layer(ego, parts, W1, b1, W2, b2):
    BR = 1000

    def body(ego_ref, s0_ref, s1_ref, W1_ref, b1_ref, W2_ref, b2_ref,
             eo_ref, no_ref):
        ego_b = ego_ref[...]
        side = s0_ref[0] + s1_ref[0]
        a = jnp.dot(ego_b + side, W1_ref[...],
                    preferred_element_type=jnp.float32) + b1_ref[...]
        sum_emb = jnp.where(a >= 0, a, 0.01 * a)
        b = jnp.dot(ego_b * side, W2_ref[...],
                    preferred_element_type=jnp.float32) + b2_ref[...]
        bi_emb = jnp.where(b >= 0, b, 0.01 * b)
        e = sum_emb + bi_emb
        eo_ref[...] = e
        n = jnp.sqrt(jnp.sum(e * e, axis=-1, keepdims=True))
        no_ref[...] = e / jnp.maximum(n, 1e-12)

    row_spec = pl.BlockSpec((BR, D), lambda i: (i, 0))
    p0_spec = pl.BlockSpec((1, BR, D), lambda i: (0, i, 0))
    p1_spec = pl.BlockSpec((1, BR, D), lambda i: (1, i, 0))
    mat_spec = pl.BlockSpec((D, D), lambda i: (0, 0))
    vec_spec = pl.BlockSpec((1, D), lambda i: (0, 0))
    return pl.pallas_call(
        body,
        grid=(N // BR,),
        in_specs=[row_spec, p0_spec, p1_spec,
                  mat_spec, vec_spec, mat_spec, vec_spec],
        out_specs=[row_spec, row_spec],
        out_shape=[jax.ShapeDtypeStruct((N, D), jnp.float32)] * 2,
    )(ego, parts, parts, W1, b1.reshape(1, D), W2, b2.reshape(1, D))


def kernel(embeddings, edge_index, edge_values,
           W1_0, b1_0, W2_0, b2_0, W1_1, b1_1, W2_1, b2_1):
    src = edge_index[0].astype(jnp.int32)
    dst = edge_index[1].astype(jnp.int32)
    vals = edge_values.astype(jnp.float32)

    parts0 = _spmm_sc(embeddings, src, dst, vals)
    ego1, norm1 = _dense_layer(embeddings, parts0, W1_0, b1_0, W2_0, b2_0)
    parts1 = _spmm_sc(norm1, src, dst, vals)
    _, norm2 = _dense_layer(ego1, parts1, W1_1, b1_1, W2_1, b2_1)
    return (embeddings, norm1, norm2)
